# baseline (speedup 1.0000x reference)
import functools

import jax
import jax.numpy as jnp
from jax import lax
from jax.experimental import pallas as pl
from jax.experimental.pallas import tpu as pltpu
from jax.experimental.pallas import tpu_sc as plsc

B = 16384
NC = 26
NN = 13
V = 100000
D = 32
NF = NC + NN

_NCORES = 2
_NSUB = 16
_NW = _NCORES * _NSUB
_BC = 4096
_NCHUNK = B // _BC
_FPW = NC // 2
_NUM_UNITS = [(j, ch) for j in range(NN) for ch in range(_NCHUNK)]


def _sc_body(xcat_hbm, xnum_hbm, table_hbm, wt_hbm, bt_hbm, out_hbm,
             planebuf, colbuf, obuf, xchunk, wtv, btv,
             psem, csem, osem):
    wid = lax.axis_index("s") * _NCORES + lax.axis_index("c")
    s_id = lax.axis_index("s")
    d0 = wid % 16
    fbase = (wid // 16) * _FPW

    pltpu.sync_copy(wt_hbm.at[pl.ds(wid, 1), :], wtv)
    pltpu.sync_copy(bt_hbm.at[pl.ds(wid, 1), :], btv)
    wvec = wtv[0, pl.ds(0, 16)]
    bvec = btv[0, pl.ds(0, 16)]
    zidx = jnp.zeros((16,), jnp.int32)

    state = {"slot": 0, "pending": [False, False]}

    def acquire_obuf():
        s = state["slot"]
        state["slot"] = 1 - s
        if state["pending"][s]:
            pltpu.make_async_copy(
                out_hbm.at[pl.ds(0, 1), pl.ds(0, 1), pl.ds(0, _BC)],
                obuf.at[s], osem).wait()
        state["pending"][s] = True
        return s

    def store_out(s, feat, d, ch):
        pltpu.async_copy(
            obuf.at[s],
            out_hbm.at[pl.ds(feat, 1), pl.ds(d, 1), pl.ds(ch * _BC, _BC)],
            osem)

    def num_unit(j, ch):
        pltpu.sync_copy(
            xnum_hbm.at[pl.ds(j, 1), pl.ds(ch * _BC, _BC)], xchunk)
        w_jd = wvec[j]
        b_jd = bvec[j]
        s = acquire_obuf()

        def proj(i, carry):
            for u in range(2):
                xv = xchunk[0, pl.ds(i * 32 + u * 16, 16)]
                obuf[s, 0, 0, pl.ds(i * 32 + u * 16, 16)] = xv * w_jd + b_jd
            return carry
        lax.fori_loop(0, _BC // 32, proj, 0)
        store_out(s, NC + j, wid, ch)

    def gather_chunk(f, d, ch):
        s = acquire_obuf()

        def gath(i, carry):
            for u in range(2):
                iv = colbuf[0, pl.ds(ch * _BC + i * 32 + u * 16, 16)]
                obuf[s, 0, 0, pl.ds(i * 32 + u * 16, 16)] = plsc.load_gather(
                    planebuf, [zidx, zidx, iv])
            return carry
        lax.fori_loop(0, _BC // 32, gath, 0)
        store_out(s, f, d, ch)

    widx = 0
    for k in range(_FPW):
        f = fbase + k
        for dd in range(2):
            d = d0 + dd * 16
            pltpu.async_copy(
                table_hbm.at[pl.ds(f, 1), pl.ds(d, 1), :], planebuf, psem)
            if dd == 0:
                pltpu.async_copy(xcat_hbm.at[pl.ds(f, 1), :], colbuf, csem)
            for j, ch in _NUM_UNITS[2 * widx:2 * widx + 2]:
                num_unit(j, ch)
            widx += 1
            pltpu.make_async_copy(
                table_hbm.at[pl.ds(f, 1), pl.ds(d, 1), :], planebuf,
                psem).wait()
            if dd == 0:
                pltpu.make_async_copy(
                    xcat_hbm.at[pl.ds(0, 1), :], colbuf, csem).wait()
            for ch in range(_NCHUNK):
                gather_chunk(f, d, ch)

    for s in range(2):
        if state["pending"][s]:
            pltpu.make_async_copy(
                out_hbm.at[pl.ds(0, 1), pl.ds(0, 1), pl.ds(0, _BC)],
                obuf.at[s], osem).wait()


@jax.jit
def _tokenize(x_catT, x_numT, tableT, wT, bT):
    mesh = plsc.VectorSubcoreMesh(core_axis_name="c", subcore_axis_name="s")
    kern = functools.partial(
        pl.kernel,
        out_type=jax.ShapeDtypeStruct((NF, D, B), jnp.float32),
        mesh=mesh,
        scratch_types=[
            pltpu.VMEM((1, 1, V), jnp.float32),
            pltpu.VMEM((1, B), jnp.int32),
            pltpu.VMEM((2, 1, 1, _BC), jnp.float32),
            pltpu.VMEM((1, _BC), jnp.float32),
            pltpu.VMEM((1, 16), jnp.float32),
            pltpu.VMEM((1, 16), jnp.float32),
            pltpu.SemaphoreType.DMA,
            pltpu.SemaphoreType.DMA,
            pltpu.SemaphoreType.DMA,
        ],
        compiler_params=pltpu.CompilerParams(
            use_tc_tiling_on_sc=True, needs_layout_passes=False),
    )(_sc_body)
    return kern(x_catT, x_numT, tableT, wT, bT)


def kernel(x_num, x_cat, cat_tables, num_W, num_b):
    tableT = jnp.transpose(cat_tables, (0, 2, 1))
    x_catT = x_cat.T.astype(jnp.int32)
    x_numT = x_num.T
    wT = jnp.pad(num_W.T, ((0, 0), (0, 16 - NN)))
    bT = jnp.pad(num_b.T, ((0, 0), (0, 16 - NN)))
    outT = _tokenize(x_catT, x_numT, tableT, wT, bT)
    return jnp.transpose(outT, (2, 0, 1))


# speedup vs baseline: 1.2162x; 1.2162x over previous
import functools

import jax
import jax.numpy as jnp
from jax import lax
from jax.experimental import pallas as pl
from jax.experimental.pallas import tpu as pltpu
from jax.experimental.pallas import tpu_sc as plsc

B = 16384
NC = 26
NN = 13
V = 100000
D = 32
NF = NC + NN

_NCORES = 2
_NSUB = 16
_NW = _NCORES * _NSUB
_BC = 4096
_NCHUNK = B // _BC
_NUM_UNITS = [(j, ch) for j in range(NN) for ch in range(_NCHUNK)]


def _sc_body(xcat_hbm, xnum_hbm, table_hbm, wt_hbm, bt_hbm, out_hbm,
             planebuf, colbuf, obuf, xchunk, wtv, btv, psem, csem, osem):
    wid = lax.axis_index("s") * _NCORES + lax.axis_index("c")
    d = wid

    pltpu.sync_copy(wt_hbm, wtv)
    pltpu.sync_copy(bt_hbm, btv)
    wvec = wtv[d, pl.ds(0, 16)]
    bvec = btv[d, pl.ds(0, 16)]
    zidx = jnp.zeros((16,), jnp.int32)

    state = {"slot": 0, "pending": [False, False]}

    def acquire_obuf():
        s = state["slot"]
        state["slot"] = 1 - s
        if state["pending"][s]:
            pltpu.make_async_copy(
                out_hbm.at[pl.ds(0, 1), pl.ds(0, 1), pl.ds(0, _BC)],
                obuf.at[s], osem).wait()
        state["pending"][s] = True
        return s

    def store_out(s, feat, ch):
        pltpu.async_copy(
            obuf.at[s],
            out_hbm.at[pl.ds(feat, 1), pl.ds(d, 1), pl.ds(ch * _BC, _BC)],
            osem)

    def fire_col(f, ch, cs):
        pltpu.async_copy(
            xcat_hbm.at[pl.ds(f, 1), pl.ds(ch * _BC, _BC)], colbuf.at[cs],
            csem)

    def wait_col(cs):
        pltpu.make_async_copy(
            xcat_hbm.at[pl.ds(0, 1), pl.ds(0, _BC)], colbuf.at[cs],
            csem).wait()

    def num_unit(j, ch):
        pltpu.sync_copy(
            xnum_hbm.at[pl.ds(j, 1), pl.ds(ch * _BC, _BC)], xchunk)
        w_jd = wvec[j]
        b_jd = bvec[j]
        s = acquire_obuf()

        def proj(i, carry):
            for u in range(2):
                xv = xchunk[0, pl.ds(i * 32 + u * 16, 16)]
                obuf[s, 0, 0, pl.ds(i * 32 + u * 16, 16)] = xv * w_jd + b_jd
            return carry
        lax.fori_loop(0, _BC // 32, proj, 0)
        store_out(s, NC + j, ch)

    def gather_chunk(f, ch, cs):
        wait_col(cs)
        if ch + 1 < _NCHUNK:
            fire_col(f, ch + 1, 1 - cs)
        s = acquire_obuf()

        def gath(i, carry):
            for u in range(2):
                iv = colbuf[cs, 0, pl.ds(i * 32 + u * 16, 16)]
                obuf[s, 0, 0, pl.ds(i * 32 + u * 16, 16)] = plsc.load_gather(
                    planebuf, [zidx, zidx, iv])
            return carry
        lax.fori_loop(0, _BC // 32, gath, 0)
        store_out(s, f, ch)

    _SPLITS = [(0, 49920), (49920, 50080)]
    for f in range(NC):
        for q0, qn in _SPLITS:
            pltpu.async_copy(
                table_hbm.at[pl.ds(f, 1), pl.ds(d, 1), pl.ds(q0, qn)],
                planebuf.at[pl.ds(0, 1), pl.ds(0, 1), pl.ds(q0, qn)],
                psem)
        fire_col(f, 0, 0)
        for j, ch in _NUM_UNITS[2 * f:2 * f + 2]:
            num_unit(j, ch)
        for q0, qn in _SPLITS:
            pltpu.make_async_copy(
                table_hbm.at[pl.ds(f, 1), pl.ds(d, 1), pl.ds(q0, qn)],
                planebuf.at[pl.ds(0, 1), pl.ds(0, 1), pl.ds(q0, qn)],
                psem).wait()
        for ch in range(_NCHUNK):
            gather_chunk(f, ch, ch % 2)

    for j, ch in _NUM_UNITS[2 * NC:]:
        num_unit(j, ch)

    for s in range(2):
        if state["pending"][s]:
            pltpu.make_async_copy(
                out_hbm.at[pl.ds(0, 1), pl.ds(0, 1), pl.ds(0, _BC)],
                obuf.at[s], osem).wait()


@jax.jit
def _tokenize(x_catT, x_numT, tableT, wT, bT):
    mesh = plsc.VectorSubcoreMesh(core_axis_name="c", subcore_axis_name="s")
    kern = functools.partial(
        pl.kernel,
        out_type=jax.ShapeDtypeStruct((NF, D, B), jnp.float32),
        mesh=mesh,
        scratch_types=[
            pltpu.VMEM((1, 1, V), jnp.float32),
            pltpu.VMEM((2, 1, _BC), jnp.int32),
            pltpu.VMEM((2, 1, 1, _BC), jnp.float32),
            pltpu.VMEM((1, _BC), jnp.float32),
            pltpu.VMEM((D, 16), jnp.float32),
            pltpu.VMEM((D, 16), jnp.float32),
            pltpu.SemaphoreType.DMA,
            pltpu.SemaphoreType.DMA,
            pltpu.SemaphoreType.DMA,
        ],
        compiler_params=pltpu.CompilerParams(
            use_tc_tiling_on_sc=True, needs_layout_passes=False),
    )(_sc_body)
    return kern(x_catT, x_numT, tableT, wT, bT)


def kernel(x_num, x_cat, cat_tables, num_W, num_b):
    tableT = jnp.transpose(cat_tables, (0, 2, 1))
    x_catT = x_cat.T.astype(jnp.int32)
    x_numT = x_num.T
    wT = jnp.pad(num_W.T, ((0, 0), (0, 16 - NN)))
    bT = jnp.pad(num_b.T, ((0, 0), (0, 16 - NN)))
    outT = _tokenize(x_catT, x_numT, tableT, wT, bT)
    return jnp.transpose(outT, (2, 0, 1))


# speedup vs baseline: 1.2333x; 1.0140x over previous
import functools

import jax
import jax.numpy as jnp
from jax import lax
from jax.experimental import pallas as pl
from jax.experimental.pallas import tpu as pltpu
from jax.experimental.pallas import tpu_sc as plsc

B = 16384
NC = 26
NN = 13
V = 100000
D = 32
NF = NC + NN

_NCORES = 2
_NSUB = 16
_NW = _NCORES * _NSUB
_BC = 4096
_NCHUNK = B // _BC
_NUM_UNITS = [(j, ch) for j in range(NN) for ch in range(_NCHUNK)]


def _sc_body(xcat_hbm, xnum_hbm, table_hbm, wt_hbm, bt_hbm, out_hbm,
             planebuf, colbuf, obuf, xchunk, wtv, btv, psem, csem, osem):
    wid = lax.axis_index("s") * _NCORES + lax.axis_index("c")
    d = wid

    pltpu.sync_copy(wt_hbm, wtv)
    pltpu.sync_copy(bt_hbm, btv)
    wvec = wtv[d, pl.ds(0, 16)]
    bvec = btv[d, pl.ds(0, 16)]
    zidx = jnp.zeros((16,), jnp.int32)

    state = {"slot": 0, "pending": [False, False]}

    def acquire_obuf():
        s = state["slot"]
        state["slot"] = 1 - s
        if state["pending"][s]:
            pltpu.make_async_copy(
                out_hbm.at[pl.ds(0, 1), pl.ds(0, 1), pl.ds(0, _BC)],
                obuf.at[s], osem).wait()
        state["pending"][s] = True
        return s

    def store_out(s, feat, ch):
        pltpu.async_copy(
            obuf.at[s],
            out_hbm.at[pl.ds(feat, 1), pl.ds(d, 1), pl.ds(ch * _BC, _BC)],
            osem)

    def fire_col(f, ch, cs):
        pltpu.async_copy(
            xcat_hbm.at[pl.ds(f, 1), pl.ds(ch * _BC, _BC)], colbuf.at[cs],
            csem)

    def wait_col(cs):
        pltpu.make_async_copy(
            xcat_hbm.at[pl.ds(0, 1), pl.ds(0, _BC)], colbuf.at[cs],
            csem).wait()

    def num_unit(j, ch):
        pltpu.sync_copy(
            xnum_hbm.at[pl.ds(j, 1), pl.ds(ch * _BC, _BC)], xchunk)
        w_jd = wvec[j]
        b_jd = bvec[j]
        s = acquire_obuf()

        def proj(i, carry):
            for u in range(4):
                xv = xchunk[0, pl.ds(i * 64 + u * 16, 16)]
                obuf[s, 0, 0, pl.ds(i * 64 + u * 16, 16)] = xv * w_jd + b_jd
            return carry
        lax.fori_loop(0, _BC // 64, proj, 0)
        store_out(s, NC + j, ch)

    def gather_chunk(f, ch, cs):
        wait_col(cs)
        if ch + 1 < _NCHUNK:
            fire_col(f, ch + 1, 1 - cs)
        s = acquire_obuf()

        def gath(i, carry):
            for u in range(4):
                iv = colbuf[cs, 0, pl.ds(i * 64 + u * 16, 16)]
                obuf[s, 0, 0, pl.ds(i * 64 + u * 16, 16)] = plsc.load_gather(
                    planebuf, [zidx, zidx, iv])
            return carry
        lax.fori_loop(0, _BC // 64, gath, 0)
        store_out(s, f, ch)

    _SPLITS = [(0, 49920), (49920, 50080)]
    for f in range(NC):
        for q0, qn in _SPLITS:
            pltpu.async_copy(
                table_hbm.at[pl.ds(f, 1), pl.ds(d, 1), pl.ds(q0, qn)],
                planebuf.at[pl.ds(0, 1), pl.ds(0, 1), pl.ds(q0, qn)],
                psem)
        fire_col(f, 0, 0)
        for j, ch in _NUM_UNITS[2 * f:2 * f + 2]:
            num_unit(j, ch)
        for q0, qn in _SPLITS:
            pltpu.make_async_copy(
                table_hbm.at[pl.ds(f, 1), pl.ds(d, 1), pl.ds(q0, qn)],
                planebuf.at[pl.ds(0, 1), pl.ds(0, 1), pl.ds(q0, qn)],
                psem).wait()
        for ch in range(_NCHUNK):
            gather_chunk(f, ch, ch % 2)

    for j, ch in _NUM_UNITS[2 * NC:]:
        num_unit(j, ch)

    for s in range(2):
        if state["pending"][s]:
            pltpu.make_async_copy(
                out_hbm.at[pl.ds(0, 1), pl.ds(0, 1), pl.ds(0, _BC)],
                obuf.at[s], osem).wait()


@jax.jit
def _tokenize(x_catT, x_numT, tableT, wT, bT):
    mesh = plsc.VectorSubcoreMesh(core_axis_name="c", subcore_axis_name="s")
    kern = functools.partial(
        pl.kernel,
        out_type=jax.ShapeDtypeStruct((NF, D, B), jnp.float32),
        mesh=mesh,
        scratch_types=[
            pltpu.VMEM((1, 1, V), jnp.float32),
            pltpu.VMEM((2, 1, _BC), jnp.int32),
            pltpu.VMEM((2, 1, 1, _BC), jnp.float32),
            pltpu.VMEM((1, _BC), jnp.float32),
            pltpu.VMEM((D, 16), jnp.float32),
            pltpu.VMEM((D, 16), jnp.float32),
            pltpu.SemaphoreType.DMA,
            pltpu.SemaphoreType.DMA,
            pltpu.SemaphoreType.DMA,
        ],
        compiler_params=pltpu.CompilerParams(
            use_tc_tiling_on_sc=True, needs_layout_passes=False),
    )(_sc_body)
    return kern(x_catT, x_numT, tableT, wT, bT)


def kernel(x_num, x_cat, cat_tables, num_W, num_b):
    tableT = jnp.transpose(cat_tables, (0, 2, 1))
    x_catT = x_cat.T.astype(jnp.int32)
    x_numT = x_num.T
    wT = jnp.pad(num_W.T, ((0, 0), (0, 16 - NN)))
    bT = jnp.pad(num_b.T, ((0, 0), (0, 16 - NN)))
    outT = _tokenize(x_catT, x_numT, tableT, wT, bT)
    return jnp.transpose(outT, (2, 0, 1))
